# baseline (speedup 1.0000x reference)
import functools

import jax
import jax.numpy as jnp
from jax import lax
from jax.experimental import pallas as pl
from jax.experimental.pallas import tpu as pltpu
from jax.experimental.pallas import tpu_sc as plsc

B, N, K, D, T = 4096, 50000, 32, 128, 2

_NC, _NS = 2, 16
_NW = _NC * _NS
_L = 16

_DN = (((1,), (1,)), ((), ()))

_TBLK = 2000


def _fuse_tables_body(e0, p0, e1, p1, wf, bf, f0o, f1o):
    a = wf[:, :D]
    bm = wf[:, D:]
    bias = bf[...]
    f0o[...] = jnp.maximum(
        lax.dot_general(e0[...], a, _DN, preferred_element_type=jnp.float32)
        + lax.dot_general(p0[...], bm, _DN, preferred_element_type=jnp.float32)
        + bias, 0.0)
    f1o[...] = jnp.maximum(
        lax.dot_general(e1[...], a, _DN, preferred_element_type=jnp.float32)
        + lax.dot_general(p1[...], bm, _DN, preferred_element_type=jnp.float32)
        + bias, 0.0)


def _fuse_tables(e0, p0, e1, p1, wf, bf2):
    tab_spec = pl.BlockSpec((_TBLK, D), lambda i: (i, 0))
    return pl.pallas_call(
        _fuse_tables_body,
        grid=(N // _TBLK,),
        in_specs=[
            tab_spec, tab_spec, tab_spec, tab_spec,
            pl.BlockSpec((D, 2 * D), lambda i: (0, 0)),
            pl.BlockSpec((1, D), lambda i: (0, 0)),
        ],
        out_specs=[tab_spec, tab_spec],
        out_shape=[
            jax.ShapeDtypeStruct((N, D), jnp.float32),
            jax.ShapeDtypeStruct((N, D), jnp.float32),
        ],
    )(e0, p0, e1, p1, wf, bf2)


_QBLK = 1024


def _fuse_q_body(ne, npf, wf, bf, qo):
    a = wf[:, :D]
    bm = wf[:, D:]
    qo[...] = jnp.maximum(
        lax.dot_general(ne[...], a, _DN, preferred_element_type=jnp.float32)
        + lax.dot_general(npf[...], bm, _DN, preferred_element_type=jnp.float32)
        + bf[...], 0.0)


def _fuse_q(ne, npf, wf, bf2):
    row_spec = pl.BlockSpec((_QBLK, D), lambda i: (i, 0))
    return pl.pallas_call(
        _fuse_q_body,
        grid=(B // _QBLK,),
        in_specs=[row_spec, row_spec,
                  pl.BlockSpec((D, 2 * D), lambda i: (0, 0)),
                  pl.BlockSpec((1, D), lambda i: (0, 0))],
        out_specs=row_spec,
        out_shape=jax.ShapeDtypeStruct((B, D), jnp.float32),
    )(ne, npf, wf, bf2)


_PWN = B // _NW


def _gather_nodes_body(nemb, nprof, nds, one, onp, idxn, rowsn, gsem):
    wid = lax.axis_index("s") * _NC + lax.axis_index("c")
    nb = wid * _PWN
    pltpu.sync_copy(nds.at[pl.ds(nb, _PWN)], idxn)
    pltpu.async_copy(nemb.at[idxn], rowsn, gsem).wait()
    pltpu.sync_copy(rowsn, one.at[pl.ds(nb, _PWN)])
    pltpu.async_copy(nprof.at[idxn], rowsn, gsem).wait()
    pltpu.sync_copy(rowsn, onp.at[pl.ds(nb, _PWN)])


@functools.cache
def _build_gather_nodes():
    return functools.partial(
        pl.kernel,
        out_type=[
            jax.ShapeDtypeStruct((B, D), jnp.float32),
            jax.ShapeDtypeStruct((B, D), jnp.float32),
        ],
        mesh=plsc.VectorSubcoreMesh(core_axis_name="c", subcore_axis_name="s"),
        scratch_types=[
            pltpu.VMEM((_PWN,), jnp.int32),
            pltpu.VMEM((_PWN, D), jnp.float32),
            pltpu.SemaphoreType.DMA,
        ],
    )(_gather_nodes_body)


_BW = B // _NW
_CB = 8
_CROWS = _CB * K
_NCH = _BW // _CB


def _attn_body(f0, idx0, f1, idx1, qn, feat0, feat1,
               qv, idxv0, idxv1, rowsv0, rowsv1, av, featv, gsem):
    wid = lax.axis_index("s") * _NC + lax.axis_index("c")
    iota = lax.iota(jnp.int32, _L)
    zeros = jnp.zeros((_L,), jnp.float32)

    pltpu.sync_copy(qn.at[pl.ds(wid * _BW, _BW)], qv)

    def splat_i(val):
        return jnp.full((_L,), val, jnp.int32)

    def type_loop(tab, idxs, feat_out):
        gbase = wid * _BW * K

        def compute_chunk(c, rowsv):
            def b_body(bi, _):
                def dg_body(dg, carry):
                    s0, s1 = carry
                    for dd in range(8):
                        d = dg * 8 + dd
                        qsp = plsc.load_gather(qv, [splat_i(c * _CB + bi),
                                                    splat_i(d)])
                        r0 = plsc.load_gather(
                            rowsv, [bi * K + iota, splat_i(d)])
                        r1 = plsc.load_gather(
                            rowsv, [bi * K + _L + iota, splat_i(d)])
                        s0 = s0 + qsp * r0
                        s1 = s1 + qsp * r1
                    return s0, s1
                s0, s1 = lax.fori_loop(0, D // 8, dg_body, (zeros, zeros))
                m = jnp.max(jnp.maximum(s0, s1))
                e0 = jnp.exp(s0 - m)
                e1 = jnp.exp(s1 - m)
                ssum = jnp.sum(e0) + jnp.sum(e1)
                av[pl.ds(0, _L)] = e0 / ssum
                av[pl.ds(_L, _L)] = e1 / ssum
                def k_body(k, facc):
                    asp = plsc.load_gather(av, [splat_i(k)])
                    return tuple(
                        facc[dg] + asp * plsc.load_gather(
                            rowsv, [splat_i(bi * K + k), dg * _L + iota])
                        for dg in range(D // _L))
                facc = lax.fori_loop(0, K, k_body, (zeros,) * (D // _L))
                for dg in range(D // _L):
                    plsc.store_scatter(
                        featv, [splat_i(bi), dg * _L + iota], facc[dg])
                return 0
            lax.fori_loop(0, _CB, b_body, 0)
            pltpu.sync_copy(
                featv, feat_out.at[pl.ds(wid * _BW + c * _CB, _CB)])

        bufs = ((idxv0, rowsv0), (idxv1, rowsv1))

        def start_gather(c, p):
            idxv, rowsv = bufs[p]
            pltpu.sync_copy(idxs.at[pl.ds(gbase + c * _CROWS, _CROWS)], idxv)
            return pltpu.async_copy(tab.at[idxv], rowsv, gsem)

        start_gather(0, 0).wait()
        def pair_body(cc, carry):
            for p in range(2):
                c = cc * 2 + p
                @pl.when(c + 1 < _NCH)
                def _():
                    idxv, rowsv = bufs[1 - p]
                    pltpu.sync_copy(
                        idxs.at[pl.ds(gbase + (c + 1) * _CROWS, _CROWS)],
                        idxv)
                    pltpu.async_copy(tab.at[idxv], rowsv, gsem)
                compute_chunk(c, bufs[p][1])
                @pl.when(c + 1 < _NCH)
                def _():
                    idxv, rowsv = bufs[1 - p]
                    pltpu.make_async_copy(tab.at[idxv], rowsv, gsem).wait()
            return carry
        lax.fori_loop(0, _NCH // 2, pair_body, 0)

    type_loop(f0, idx0, feat0)
    type_loop(f1, idx1, feat1)


@functools.cache
def _build_attn():
    return functools.partial(
        pl.kernel,
        out_type=[
            jax.ShapeDtypeStruct((B, D), jnp.float32),
            jax.ShapeDtypeStruct((B, D), jnp.float32),
        ],
        mesh=plsc.VectorSubcoreMesh(core_axis_name="c", subcore_axis_name="s"),
        compiler_params=pltpu.CompilerParams(needs_layout_passes=False),
        scratch_types=[
            pltpu.VMEM((_BW, D), jnp.float32),
            pltpu.VMEM((_CROWS,), jnp.int32),
            pltpu.VMEM((_CROWS,), jnp.int32),
            pltpu.VMEM((_CROWS, D), jnp.float32),
            pltpu.VMEM((_CROWS, D), jnp.float32),
            pltpu.VMEM((K,), jnp.float32),
            pltpu.VMEM((_CB, D), jnp.float32),
            pltpu.SemaphoreType.DMA,
        ],
    )(_attn_body)


_BB = 1024


def _tail_body(qr, f0r, f1r, w1, b1, w2, b2, w, bb, wt, combo, atto):
    agg0 = jnp.maximum(
        lax.dot_general(f0r[...], w1[...], _DN,
                        preferred_element_type=jnp.float32) + b1[...], 0.0)
    agg1 = jnp.maximum(
        lax.dot_general(f1r[...], w1[...], _DN,
                        preferred_element_type=jnp.float32) + b1[...], 0.0)
    ta = jnp.concatenate([agg0, agg1], axis=1)
    mta = lax.dot_general(ta, wt[...], _DN, preferred_element_type=jnp.float32)
    mm = jnp.max(mta, axis=1, keepdims=True)
    ee = jnp.exp(mta - mm)
    att = ee / jnp.sum(ee, axis=1, keepdims=True)
    fin = att[:, 0:1] * agg0 + att[:, 1:2] * agg1
    fin = jnp.maximum(
        lax.dot_general(fin, w2[...], _DN, preferred_element_type=jnp.float32)
        + b2[...], 0.0)
    comb = jnp.concatenate([qr[...], fin], axis=1)
    combo[...] = jnp.maximum(
        lax.dot_general(comb, w[...], _DN, preferred_element_type=jnp.float32)
        + bb[...], 0.0)
    atto[...] = att


def _tail(qn, f0, f1, w1, b12, w2, b22, w, b2d, wt):
    row_spec = pl.BlockSpec((_BB, D), lambda i: (i, 0))
    full = lambda shape: pl.BlockSpec(shape, lambda i: tuple(0 for _ in shape))
    return pl.pallas_call(
        _tail_body,
        grid=(B // _BB,),
        in_specs=[row_spec, row_spec, row_spec,
                  full((D, D)), full((1, D)),
                  full((D, D)), full((1, D)),
                  full((D, 2 * D)), full((1, D)),
                  full((T, 2 * D))],
        out_specs=[row_spec, pl.BlockSpec((_BB, T), lambda i: (i, 0))],
        out_shape=[
            jax.ShapeDtypeStruct((B, D), jnp.float32),
            jax.ShapeDtypeStruct((B, T), jnp.float32),
        ],
    )(qn, f0, f1, w1, b12, w2, b22, w, b2d, wt)


def kernel(nodes, neigh_idx_0, neigh_idx_1, node_emb, node_prof,
           neigh_emb_0, neigh_prof_0, neigh_emb_1, neigh_prof_1,
           Wf, bf, W1, b1, W2, b2, W, b, Wt):
    nodes_i = nodes.astype(jnp.int32)
    idx0 = neigh_idx_0.astype(jnp.int32).reshape(-1)
    idx1 = neigh_idx_1.astype(jnp.int32).reshape(-1)

    bf2 = bf.reshape(1, D)
    ne, npf = _build_gather_nodes()(node_emb, node_prof, nodes_i)
    qn = _fuse_q(ne, npf, Wf, bf2)
    f0, f1 = _fuse_tables(neigh_emb_0, neigh_prof_0, neigh_emb_1,
                          neigh_prof_1, Wf, bf2)
    feat0, feat1 = _build_attn()(f0, idx0, f1, idx1, qn)
    comb, att = _tail(qn, feat0, feat1, W1, b1.reshape(1, D),
                      W2, b2.reshape(1, D), W, b.reshape(1, D), Wt)
    return comb, att.reshape(B, T, 1)


# speedup vs baseline: 2.5246x; 2.5246x over previous
import functools

import jax
import jax.numpy as jnp
from jax import lax
from jax.experimental import pallas as pl
from jax.experimental.pallas import tpu as pltpu
from jax.experimental.pallas import tpu_sc as plsc

B, N, K, D, T = 4096, 50000, 32, 128, 2

_NC, _NS = 2, 16
_NW = _NC * _NS

_DN = (((1,), (1,)), ((), ()))

_TBLK = 2000


def _fuse_table_body(e, p, wf, bf, fo):
    fo[...] = jnp.maximum(
        lax.dot_general(e[...], wf[:, :D], _DN,
                        preferred_element_type=jnp.float32)
        + lax.dot_general(p[...], wf[:, D:], _DN,
                          preferred_element_type=jnp.float32)
        + bf[...], 0.0)


def _fuse_table(e, p, wf, bf2):
    tab_spec = pl.BlockSpec((_TBLK, D), lambda i: (i, 0))
    return pl.pallas_call(
        _fuse_table_body,
        grid=(N // _TBLK,),
        in_specs=[tab_spec, tab_spec,
                  pl.BlockSpec((D, 2 * D), lambda i: (0, 0)),
                  pl.BlockSpec((1, D), lambda i: (0, 0))],
        out_specs=tab_spec,
        out_shape=jax.ShapeDtypeStruct((N, D), jnp.float32),
    )(e, p, wf, bf2)


_PW = (K * B) // _NW
_C = 256
_NCHUNK = _PW // _C
_PWN = B // _NW


def _ring_gather(wid, tab, idxs, out, bufs, gsem, wsem):
    def pair_body(jj, carry):
        for p in range(2):
            j = jj * 2 + p
            base = wid * _PW + j * _C
            idxv, rowsv = bufs[p]

            @pl.when(jj > 0)
            def _drain():
                pltpu.make_async_copy(
                    rowsv, out.at[pl.ds(base - 2 * _C, _C)], wsem).wait()

            pltpu.sync_copy(idxs.at[pl.ds(base, _C)], idxv)
            pltpu.async_copy(tab.at[idxv], rowsv, gsem).wait()
            pltpu.async_copy(rowsv, out.at[pl.ds(base, _C)], wsem)
        return carry
    lax.fori_loop(0, _NCHUNK // 2, pair_body, 0)
    for p in range(2):
        base = wid * _PW + (_NCHUNK - 2 + p) * _C
        pltpu.make_async_copy(bufs[p][1], out.at[pl.ds(base, _C)],
                              wsem).wait()


def _gather_t0_body(f0, idx0, nemb, nprof, nds, out0, one, onp,
                    idxv0, idxv1, rowsv0, rowsv1, idxn, rowsn, gsem, wsem):
    wid = lax.axis_index("s") * _NC + lax.axis_index("c")
    _ring_gather(wid, f0, idx0, out0,
                 ((idxv0, rowsv0), (idxv1, rowsv1)), gsem, wsem)
    nb = wid * _PWN
    pltpu.sync_copy(nds.at[pl.ds(nb, _PWN)], idxn)
    pltpu.async_copy(nemb.at[idxn], rowsn, gsem).wait()
    pltpu.sync_copy(rowsn, one.at[pl.ds(nb, _PWN)])
    pltpu.async_copy(nprof.at[idxn], rowsn, gsem).wait()
    pltpu.sync_copy(rowsn, onp.at[pl.ds(nb, _PWN)])


def _gather_t1_body(f1, idx1, out1,
                    idxv0, idxv1, rowsv0, rowsv1, gsem, wsem):
    wid = lax.axis_index("s") * _NC + lax.axis_index("c")
    _ring_gather(wid, f1, idx1, out1,
                 ((idxv0, rowsv0), (idxv1, rowsv1)), gsem, wsem)


_RING_SCRATCH = [
    pltpu.VMEM((_C,), jnp.int32),
    pltpu.VMEM((_C,), jnp.int32),
    pltpu.VMEM((_C, D), jnp.float32),
    pltpu.VMEM((_C, D), jnp.float32),
]


@functools.cache
def _build_gather_t0():
    return functools.partial(
        pl.kernel,
        out_type=[
            jax.ShapeDtypeStruct((K * B, D), jnp.float32),
            jax.ShapeDtypeStruct((B, D), jnp.float32),
            jax.ShapeDtypeStruct((B, D), jnp.float32),
        ],
        mesh=plsc.VectorSubcoreMesh(core_axis_name="c", subcore_axis_name="s"),
        scratch_types=_RING_SCRATCH + [
            pltpu.VMEM((_PWN,), jnp.int32),
            pltpu.VMEM((_PWN, D), jnp.float32),
            pltpu.SemaphoreType.DMA,
            pltpu.SemaphoreType.DMA,
        ],
    )(_gather_t0_body)


@functools.cache
def _build_gather_t1():
    return functools.partial(
        pl.kernel,
        out_type=jax.ShapeDtypeStruct((K * B, D), jnp.float32),
        mesh=plsc.VectorSubcoreMesh(core_axis_name="c", subcore_axis_name="s"),
        scratch_types=_RING_SCRATCH + [
            pltpu.SemaphoreType.DMA,
            pltpu.SemaphoreType.DMA,
        ],
    )(_gather_t1_body)


_BB = 256


def _attention(q, nf_ref, w1v, b1v):
    ones_dk = jnp.ones((D, K), jnp.float32)
    kiota = lax.broadcasted_iota(jnp.int32, (1, K), 1)
    ones_1d = jnp.ones((1, D), jnp.float32)
    dn_nt = (((1,), (0,)), ((), ()))
    s = jnp.zeros((_BB, K), jnp.float32)
    for k in range(K):
        s = s + lax.dot_general(
            q * nf_ref[k], ones_dk * (kiota == k).astype(jnp.float32),
            dn_nt, preferred_element_type=jnp.float32)
    m = jnp.max(s, axis=1, keepdims=True)
    e = jnp.exp(s - m)
    att_k = e / jnp.sum(e, axis=1, keepdims=True)
    feat = jnp.zeros((_BB, D), jnp.float32)
    for k in range(K):
        ab = lax.dot_general(att_k[:, k:k + 1], ones_1d, dn_nt,
                             preferred_element_type=jnp.float32)
        feat = feat + ab * nf_ref[k]
    return jnp.maximum(
        lax.dot_general(feat, w1v, _DN, preferred_element_type=jnp.float32)
        + b1v, 0.0)


def _attend_t0_body(ne, npf, nf0, wf, bf, w1, b1, qo, agg0o):
    q = jnp.maximum(
        lax.dot_general(ne[...], wf[:, :D], _DN,
                        preferred_element_type=jnp.float32)
        + lax.dot_general(npf[...], wf[:, D:], _DN,
                          preferred_element_type=jnp.float32)
        + bf[...], 0.0)
    qo[...] = q
    agg0o[...] = _attention(q, nf0, w1[...], b1[...])


def _attend_t0(ne, npf, nf0, wf, bf2, w1, b12):
    row_spec = pl.BlockSpec((_BB, D), lambda i: (i, 0))
    nf_spec = pl.BlockSpec((K, _BB, D), lambda i: (0, i, 0))
    full = lambda shape: pl.BlockSpec(shape, lambda i: tuple(0 for _ in shape))
    return pl.pallas_call(
        _attend_t0_body,
        grid=(B // _BB,),
        in_specs=[row_spec, row_spec, nf_spec,
                  full((D, 2 * D)), full((1, D)),
                  full((D, D)), full((1, D))],
        out_specs=[row_spec, row_spec],
        out_shape=[
            jax.ShapeDtypeStruct((B, D), jnp.float32),
            jax.ShapeDtypeStruct((B, D), jnp.float32),
        ],
    )(ne, npf, nf0, wf, bf2, w1, b12)


def _attend_t1_body(qr, agg0r, nf1, w1, b1, w2, b2, w, bb, wt, combo, atto):
    q = qr[...]
    agg0 = agg0r[...]
    agg1 = _attention(q, nf1, w1[...], b1[...])
    ta = jnp.concatenate([agg0, agg1], axis=1)
    mta = lax.dot_general(ta, wt[...], _DN, preferred_element_type=jnp.float32)
    mm = jnp.max(mta, axis=1, keepdims=True)
    ee = jnp.exp(mta - mm)
    att = ee / jnp.sum(ee, axis=1, keepdims=True)
    fin = att[:, 0:1] * agg0 + att[:, 1:2] * agg1
    fin = jnp.maximum(
        lax.dot_general(fin, w2[...], _DN, preferred_element_type=jnp.float32)
        + b2[...], 0.0)
    comb = jnp.concatenate([q, fin], axis=1)
    combo[...] = jnp.maximum(
        lax.dot_general(comb, w[...], _DN, preferred_element_type=jnp.float32)
        + bb[...], 0.0)
    atto[...] = att


def _attend_t1(qn, agg0, nf1, w1, b12, w2, b22, w, b2d, wt):
    row_spec = pl.BlockSpec((_BB, D), lambda i: (i, 0))
    nf_spec = pl.BlockSpec((K, _BB, D), lambda i: (0, i, 0))
    full = lambda shape: pl.BlockSpec(shape, lambda i: tuple(0 for _ in shape))
    return pl.pallas_call(
        _attend_t1_body,
        grid=(B // _BB,),
        in_specs=[row_spec, row_spec, nf_spec,
                  full((D, D)), full((1, D)),
                  full((D, D)), full((1, D)),
                  full((D, 2 * D)), full((1, D)),
                  full((T, 2 * D))],
        out_specs=[row_spec, pl.BlockSpec((_BB, T), lambda i: (i, 0))],
        out_shape=[
            jax.ShapeDtypeStruct((B, D), jnp.float32),
            jax.ShapeDtypeStruct((B, T), jnp.float32),
        ],
    )(qn, agg0, nf1, w1, b12, w2, b22, w, b2d, wt)


def kernel(nodes, neigh_idx_0, neigh_idx_1, node_emb, node_prof,
           neigh_emb_0, neigh_prof_0, neigh_emb_1, neigh_prof_1,
           Wf, bf, W1, b1, W2, b2, W, b, Wt):
    nodes_i = nodes.astype(jnp.int32)
    idx0t = neigh_idx_0.astype(jnp.int32).T.reshape(-1)
    idx1t = neigh_idx_1.astype(jnp.int32).T.reshape(-1)
    bf2 = bf.reshape(1, D)

    f0 = _fuse_table(neigh_emb_0, neigh_prof_0, Wf, bf2)
    nf0, ne, npf = _build_gather_t0()(f0, idx0t, node_emb, node_prof,
                                      nodes_i)
    f1 = _fuse_table(neigh_emb_1, neigh_prof_1, Wf, bf2)
    qn, agg0 = _attend_t0(ne, npf, nf0.reshape(K, B, D), Wf, bf2,
                          W1, b1.reshape(1, D))
    nf1 = _build_gather_t1()(f1, idx1t)
    comb, att = _attend_t1(qn, agg0, nf1.reshape(K, B, D),
                           W1, b1.reshape(1, D), W2, b2.reshape(1, D),
                           W, b.reshape(1, D), Wt)
    return comb, att.reshape(B, T, 1)


# speedup vs baseline: 2.5892x; 1.0256x over previous
import functools

import jax
import jax.numpy as jnp
from jax import lax
from jax.experimental import pallas as pl
from jax.experimental.pallas import tpu as pltpu
from jax.experimental.pallas import tpu_sc as plsc

B, N, K, D, T = 4096, 50000, 32, 128, 2

_NC, _NS = 2, 16
_NW = _NC * _NS

_DN = (((1,), (1,)), ((), ()))

_TBLK = 2000


def _fuse_table_body(e, p, wf, bf, fo):
    fo[...] = jnp.maximum(
        lax.dot_general(e[...], wf[:, :D], _DN,
                        preferred_element_type=jnp.float32)
        + lax.dot_general(p[...], wf[:, D:], _DN,
                          preferred_element_type=jnp.float32)
        + bf[...], 0.0)


def _fuse_table(e, p, wf, bf2):
    tab_spec = pl.BlockSpec((_TBLK, D), lambda i: (i, 0))
    return pl.pallas_call(
        _fuse_table_body,
        grid=(N // _TBLK,),
        in_specs=[tab_spec, tab_spec,
                  pl.BlockSpec((D, 2 * D), lambda i: (0, 0)),
                  pl.BlockSpec((1, D), lambda i: (0, 0))],
        out_specs=tab_spec,
        out_shape=jax.ShapeDtypeStruct((N, D), jnp.float32),
    )(e, p, wf, bf2)


_PW = (K * B) // _NW
_C = 256
_NCHUNK = _PW // _C
_PWN = B // _NW


def _ring_gather(wid, tab, idxs, out, bufs, gsem, wsem):
    def pair_body(jj, carry):
        for p in range(2):
            j = jj * 2 + p
            base = wid * _PW + j * _C
            idxv, rowsv = bufs[p]

            @pl.when(jj > 0)
            def _drain():
                pltpu.make_async_copy(
                    rowsv, out.at[pl.ds(base - 2 * _C, _C)], wsem).wait()

            pltpu.sync_copy(idxs.at[pl.ds(base, _C)], idxv)
            pltpu.async_copy(tab.at[idxv], rowsv, gsem).wait()
            pltpu.async_copy(rowsv, out.at[pl.ds(base, _C)], wsem)
        return carry
    lax.fori_loop(0, _NCHUNK // 2, pair_body, 0)
    for p in range(2):
        base = wid * _PW + (_NCHUNK - 2 + p) * _C
        pltpu.make_async_copy(bufs[p][1], out.at[pl.ds(base, _C)],
                              wsem).wait()


def _gather_t0_body(f0, idx0, nemb, nprof, nds, out0, one, onp,
                    idxv0, idxv1, rowsv0, rowsv1, idxn, rowsn, gsem, wsem):
    wid = lax.axis_index("s") * _NC + lax.axis_index("c")
    _ring_gather(wid, f0, idx0, out0,
                 ((idxv0, rowsv0), (idxv1, rowsv1)), gsem, wsem)
    nb = wid * _PWN
    pltpu.sync_copy(nds.at[pl.ds(nb, _PWN)], idxn)
    pltpu.async_copy(nemb.at[idxn], rowsn, gsem).wait()
    pltpu.sync_copy(rowsn, one.at[pl.ds(nb, _PWN)])
    pltpu.async_copy(nprof.at[idxn], rowsn, gsem).wait()
    pltpu.sync_copy(rowsn, onp.at[pl.ds(nb, _PWN)])


def _gather_t1_body(f1, idx1, out1,
                    idxv0, idxv1, rowsv0, rowsv1, gsem, wsem):
    wid = lax.axis_index("s") * _NC + lax.axis_index("c")
    _ring_gather(wid, f1, idx1, out1,
                 ((idxv0, rowsv0), (idxv1, rowsv1)), gsem, wsem)


_RING_SCRATCH = [
    pltpu.VMEM((_C,), jnp.int32),
    pltpu.VMEM((_C,), jnp.int32),
    pltpu.VMEM((_C, D), jnp.float32),
    pltpu.VMEM((_C, D), jnp.float32),
]


@functools.cache
def _build_gather_t0():
    return functools.partial(
        pl.kernel,
        out_type=[
            jax.ShapeDtypeStruct((K * B, D), jnp.float32),
            jax.ShapeDtypeStruct((B, D), jnp.float32),
            jax.ShapeDtypeStruct((B, D), jnp.float32),
        ],
        mesh=plsc.VectorSubcoreMesh(core_axis_name="c", subcore_axis_name="s"),
        scratch_types=_RING_SCRATCH + [
            pltpu.VMEM((_PWN,), jnp.int32),
            pltpu.VMEM((_PWN, D), jnp.float32),
            pltpu.SemaphoreType.DMA,
            pltpu.SemaphoreType.DMA,
        ],
    )(_gather_t0_body)


@functools.cache
def _build_gather_t1():
    return functools.partial(
        pl.kernel,
        out_type=jax.ShapeDtypeStruct((K * B, D), jnp.float32),
        mesh=plsc.VectorSubcoreMesh(core_axis_name="c", subcore_axis_name="s"),
        scratch_types=_RING_SCRATCH + [
            pltpu.SemaphoreType.DMA,
            pltpu.SemaphoreType.DMA,
        ],
    )(_gather_t1_body)


_BB = 512


def _attention(q, nf_ref, w1v, b1v):
    ones_dk = jnp.ones((D, K), jnp.float32)
    kiota = lax.broadcasted_iota(jnp.int32, (1, K), 1)
    ones_1d = jnp.ones((1, D), jnp.float32)
    dn_nt = (((1,), (0,)), ((), ()))
    s = jnp.zeros((_BB, K), jnp.float32)
    for k in range(K):
        s = s + lax.dot_general(
            q * nf_ref[k], ones_dk * (kiota == k).astype(jnp.float32),
            dn_nt, preferred_element_type=jnp.float32)
    m = jnp.max(s, axis=1, keepdims=True)
    e = jnp.exp(s - m)
    att_k = e / jnp.sum(e, axis=1, keepdims=True)
    feat = jnp.zeros((_BB, D), jnp.float32)
    for k in range(K):
        ab = lax.dot_general(att_k[:, k:k + 1], ones_1d, dn_nt,
                             preferred_element_type=jnp.float32)
        feat = feat + ab * nf_ref[k]
    return jnp.maximum(
        lax.dot_general(feat, w1v, _DN, preferred_element_type=jnp.float32)
        + b1v, 0.0)


def _attend_t0_body(ne, npf, nf0, wf, bf, w1, b1, qo, agg0o):
    q = jnp.maximum(
        lax.dot_general(ne[...], wf[:, :D], _DN,
                        preferred_element_type=jnp.float32)
        + lax.dot_general(npf[...], wf[:, D:], _DN,
                          preferred_element_type=jnp.float32)
        + bf[...], 0.0)
    qo[...] = q
    agg0o[...] = _attention(q, nf0, w1[...], b1[...])


def _attend_t0(ne, npf, nf0, wf, bf2, w1, b12):
    row_spec = pl.BlockSpec((_BB, D), lambda i: (i, 0))
    nf_spec = pl.BlockSpec((K, _BB, D), lambda i: (0, i, 0))
    full = lambda shape: pl.BlockSpec(shape, lambda i: tuple(0 for _ in shape))
    return pl.pallas_call(
        _attend_t0_body,
        grid=(B // _BB,),
        in_specs=[row_spec, row_spec, nf_spec,
                  full((D, 2 * D)), full((1, D)),
                  full((D, D)), full((1, D))],
        out_specs=[row_spec, row_spec],
        out_shape=[
            jax.ShapeDtypeStruct((B, D), jnp.float32),
            jax.ShapeDtypeStruct((B, D), jnp.float32),
        ],
    )(ne, npf, nf0, wf, bf2, w1, b12)


def _attend_t1_body(qr, agg0r, nf1, w1, b1, w2, b2, w, bb, wt, combo, atto):
    q = qr[...]
    agg0 = agg0r[...]
    agg1 = _attention(q, nf1, w1[...], b1[...])
    ta = jnp.concatenate([agg0, agg1], axis=1)
    mta = lax.dot_general(ta, wt[...], _DN, preferred_element_type=jnp.float32)
    mm = jnp.max(mta, axis=1, keepdims=True)
    ee = jnp.exp(mta - mm)
    att = ee / jnp.sum(ee, axis=1, keepdims=True)
    fin = att[:, 0:1] * agg0 + att[:, 1:2] * agg1
    fin = jnp.maximum(
        lax.dot_general(fin, w2[...], _DN, preferred_element_type=jnp.float32)
        + b2[...], 0.0)
    comb = jnp.concatenate([q, fin], axis=1)
    combo[...] = jnp.maximum(
        lax.dot_general(comb, w[...], _DN, preferred_element_type=jnp.float32)
        + bb[...], 0.0)
    atto[...] = att


def _attend_t1(qn, agg0, nf1, w1, b12, w2, b22, w, b2d, wt):
    row_spec = pl.BlockSpec((_BB, D), lambda i: (i, 0))
    nf_spec = pl.BlockSpec((K, _BB, D), lambda i: (0, i, 0))
    full = lambda shape: pl.BlockSpec(shape, lambda i: tuple(0 for _ in shape))
    return pl.pallas_call(
        _attend_t1_body,
        grid=(B // _BB,),
        in_specs=[row_spec, row_spec, nf_spec,
                  full((D, D)), full((1, D)),
                  full((D, D)), full((1, D)),
                  full((D, 2 * D)), full((1, D)),
                  full((T, 2 * D))],
        out_specs=[row_spec, pl.BlockSpec((_BB, T), lambda i: (i, 0))],
        out_shape=[
            jax.ShapeDtypeStruct((B, D), jnp.float32),
            jax.ShapeDtypeStruct((B, T), jnp.float32),
        ],
    )(qn, agg0, nf1, w1, b12, w2, b22, w, b2d, wt)


def kernel(nodes, neigh_idx_0, neigh_idx_1, node_emb, node_prof,
           neigh_emb_0, neigh_prof_0, neigh_emb_1, neigh_prof_1,
           Wf, bf, W1, b1, W2, b2, W, b, Wt):
    nodes_i = nodes.astype(jnp.int32)
    idx0t = neigh_idx_0.astype(jnp.int32).T.reshape(-1)
    idx1t = neigh_idx_1.astype(jnp.int32).T.reshape(-1)
    bf2 = bf.reshape(1, D)

    f0 = _fuse_table(neigh_emb_0, neigh_prof_0, Wf, bf2)
    nf0, ne, npf = _build_gather_t0()(f0, idx0t, node_emb, node_prof,
                                      nodes_i)
    f1 = _fuse_table(neigh_emb_1, neigh_prof_1, Wf, bf2)
    qn, agg0 = _attend_t0(ne, npf, nf0.reshape(K, B, D), Wf, bf2,
                          W1, b1.reshape(1, D))
    nf1 = _build_gather_t1()(f1, idx1t)
    comb, att = _attend_t1(qn, agg0, nf1.reshape(K, B, D),
                           W1, b1.reshape(1, D), W2, b2.reshape(1, D),
                           W, b.reshape(1, D), Wt)
    return comb, att.reshape(B, T, 1)
